# Initial kernel scaffold; baseline (speedup 1.0000x reference)
import jax
import jax.numpy as jnp
from jax.experimental import pallas as pl


def kernel(x, edges, W1, att_src1, att_dst1, bias1, W2, att_src2, att_dst2, bias2):
    raise NotImplementedError("write your pallas kernel here")


# speedup vs baseline: 53.5153x; 53.5153x over previous
import functools

import jax
import jax.numpy as jnp
from jax import lax
from jax.experimental import pallas as pl
from jax.experimental.pallas import tpu as pltpu
from jax.experimental.pallas import tpu_sc as plsc

N = 10000
FEATS = 128
HID = 64
HEADS = 8
DH = 8
CLASSES = 40
CP = 48

NC = 2
NS = 16
NW = NC * NS
L = 16

NP = 10240
STRIPE = NP // NS
B = 128
NCH = 81
C = NCH * B
E2P = NW * C

_F32 = jnp.float32
_I32 = jnp.int32


def _iota16():
    return lax.iota(_I32, L)


def _vperm(v, idx):
    dn = lax.GatherDimensionNumbers(
        offset_dims=(), collapsed_slice_dims=(0,), start_index_map=(0,))
    return lax.gather(v, idx[:, None], dn, (1,),
                      mode=lax.GatherScatterMode.PROMISE_IN_BOUNDS)


def _tc1_body(xp_ref, w1_ref, amap_s_ref, amap_d_ref,
              h1_ref, asd_ref, add_ref, m1_ref):
    h = jnp.dot(xp_ref[...], w1_ref[...], preferred_element_type=_F32)
    h1_ref[...] = h
    a_s = jnp.dot(h, amap_s_ref[...], preferred_element_type=_F32)
    a_d = jnp.dot(h, amap_d_ref[...], preferred_element_type=_F32)
    asd_ref[...] = jnp.concatenate([a_s, a_s], axis=1)
    add_ref[...] = jnp.concatenate([a_d, a_d], axis=1)
    m = jnp.maximum(jnp.max(a_s, axis=0) + jnp.max(a_d, axis=0), 0.0)
    m1_ref[...] = jnp.concatenate([m, m], axis=0)


def _tc2_body(den_ref, out_ref, b1_ref, w2_ref, as2w_ref, ad2w_ref, e8_ref,
              h2p_ref, as2_ref, ad2_ref, m2_ref):
    den = den_ref[0, :, :HEADS] + den_ref[1, :, :HEADS]
    agg = out_ref[0] + out_ref[1]
    inv = 1.0 / (den + 1e-16)
    invx = jnp.dot(inv, e8_ref[...], preferred_element_type=_F32)
    h1f = agg * invx + b1_ref[...][None, :]
    rowid = lax.broadcasted_iota(_I32, (NP, 1), 0)
    h1f = jnp.where(rowid < N, h1f, 0.0)
    h2 = jnp.dot(h1f, w2_ref[...], preferred_element_type=_F32)
    h2p_ref[...] = jnp.pad(h2, ((0, 0), (0, CP - CLASSES)))
    a_s = jnp.dot(h2, as2w_ref[...].reshape(CLASSES, 1),
                  preferred_element_type=_F32)
    a_d = jnp.dot(h2, ad2w_ref[...].reshape(CLASSES, 1),
                  preferred_element_type=_F32)
    a_s = jnp.where(rowid < N, a_s, 0.0)
    a_d = jnp.where(rowid < N, a_d, 0.0)
    as2_ref[...] = a_s
    ad2_ref[...] = a_d
    m2 = jnp.maximum(jnp.max(a_s) + jnp.max(a_d), 0.0)
    m2_ref[...] = jnp.full((L,), m2, dtype=_F32)


def _tc3_body(den_ref, out_ref, b2_ref, y_ref):
    den = den_ref[0] + den_ref[1]
    agg = out_ref[0, :, :CLASSES] + out_ref[1, :, :CLASSES]
    inv = 1.0 / (den + 1e-16)
    y = agg * inv[:, None] + b2_ref[...][None, :]
    y_ref[...] = y[:N, :]


def _sc1_body(src_hbm, dst_hbm, asd_hbm, add_hbm, h1_hbm, m1_hbm,
              den_out, msg_out,
              sidx, didx, sbuf, dbuf, exb, hbuf, msgb, mvec,
              den_sh, out_sh):
    c = lax.axis_index("c")
    s = lax.axis_index("s")
    w = c * NS + s

    def _z16(i, _):
        exb[i, :] = jnp.zeros((L,), _F32)
        return 0
    lax.fori_loop(0, B, _z16, 0)

    def _z64(i, _):
        for q in range(4):
            msgb[i, pl.ds(q * L, L)] = jnp.zeros((L,), _F32)
        return 0
    lax.fori_loop(0, B, _z64, 0)

    for r in range(STRIPE // B):
        row = s * STRIPE + r * B
        pltpu.sync_copy(exb, den_sh.at[pl.ds(row, B)])
        pltpu.sync_copy(msgb, out_sh.at[pl.ds(row, B)])
    plsc.subcore_barrier()

    pltpu.sync_copy(src_hbm.at[w], sidx)
    pltpu.sync_copy(dst_hbm.at[w], didx)
    pltpu.sync_copy(m1_hbm, mvec)

    def _chunk(ch, _):
        si = sidx.at[ch]
        di = didx.at[ch]
        pltpu.sync_copy(asd_hbm.at[si], sbuf)
        pltpu.sync_copy(add_hbm.at[di], dbuf)
        pltpu.sync_copy(h1_hbm.at[si], hbuf)
        m = mvec[...]
        half = lax.shift_right_logical(_iota16(), 3)

        def _edge(e, _):
            t = sbuf[e, :] + dbuf[e, :]
            a = jnp.maximum(t, 0.2 * t)
            ex = jnp.exp(a - m)
            exb[e, :] = ex
            for q in range(4):
                wv = _vperm(ex, half + (2 * q))
                msgb[e, pl.ds(q * L, L)] = wv * hbuf[e, pl.ds(q * L, L)]
            return 0

        lax.fori_loop(0, B, _edge, 0)
        pltpu.sync_copy(exb, den_sh.at[di], add=True)
        pltpu.sync_copy(msgb, out_sh.at[di], add=True)
        return 0

    lax.fori_loop(0, NCH, _chunk, 0)
    plsc.subcore_barrier()

    row = s * STRIPE
    pltpu.sync_copy(den_sh.at[pl.ds(row, STRIPE)],
                    den_out.at[c].at[pl.ds(row, STRIPE)])
    pltpu.sync_copy(out_sh.at[pl.ds(row, STRIPE)],
                    msg_out.at[c].at[pl.ds(row, STRIPE)])


def _sc2_body(src_hbm, dst_hbm, as2_hbm, ad2_hbm, h2_hbm, m2_hbm,
              den_out, msg_out,
              sidx, didx, astab, adtab, exb, hbuf, msgb, mvec,
              den_sh, out_sh):
    c = lax.axis_index("c")
    s = lax.axis_index("s")
    w = c * NS + s

    def _z1(i, _):
        exb[pl.ds(i * L, L)] = jnp.zeros((L,), _F32)
        return 0
    lax.fori_loop(0, B // L, _z1, 0)

    def _z48(i, _):
        for q in range(3):
            msgb[i, pl.ds(q * L, L)] = jnp.zeros((L,), _F32)
        return 0
    lax.fori_loop(0, B, _z48, 0)

    for r in range(STRIPE // B):
        row = s * STRIPE + r * B
        pltpu.sync_copy(exb, den_sh.at[pl.ds(row, B)])
        pltpu.sync_copy(msgb, out_sh.at[pl.ds(row, B)])
    plsc.subcore_barrier()

    pltpu.sync_copy(src_hbm.at[w], sidx)
    pltpu.sync_copy(dst_hbm.at[w], didx)
    pltpu.sync_copy(m2_hbm, mvec)
    pltpu.sync_copy(as2_hbm, astab)
    pltpu.sync_copy(ad2_hbm, adtab)

    def _chunk(ch, _):
        si = sidx.at[ch]
        di = didx.at[ch]
        pltpu.sync_copy(h2_hbm.at[si], hbuf)
        m = mvec[...]

        def _grp(g, _):
            sv = sidx[ch, pl.ds(g * L, L)]
            dv = didx[ch, pl.ds(g * L, L)]
            t = plsc.load_gather(astab, [sv]) + plsc.load_gather(adtab, [dv])
            a = jnp.maximum(t, 0.2 * t)
            exb[pl.ds(g * L, L)] = jnp.exp(a - m)
            return 0

        lax.fori_loop(0, B // L, _grp, 0)

        def _grp2(g, _):
            exg = exb[pl.ds(g * L, L)]

            def _edge(i, _):
                e = g * L + i
                wv = _vperm(exg, jnp.full((L,), i, dtype=_I32))
                for q in range(3):
                    msgb[e, pl.ds(q * L, L)] = wv * hbuf[e, pl.ds(q * L, L)]
                return 0

            lax.fori_loop(0, L, _edge, 0)
            return 0

        lax.fori_loop(0, B // L, _grp2, 0)
        pltpu.sync_copy(exb, den_sh.at[di], add=True)
        pltpu.sync_copy(msgb, out_sh.at[di], add=True)
        return 0

    lax.fori_loop(0, NCH, _chunk, 0)
    plsc.subcore_barrier()

    row = s * STRIPE
    pltpu.sync_copy(den_sh.at[pl.ds(row, STRIPE)],
                    den_out.at[c].at[pl.ds(row, STRIPE)])
    pltpu.sync_copy(out_sh.at[pl.ds(row, STRIPE)],
                    msg_out.at[c].at[pl.ds(row, STRIPE)])


@functools.lru_cache(maxsize=1)
def _make_kernels():
    tc1 = pl.pallas_call(
        _tc1_body,
        out_shape=[
            jax.ShapeDtypeStruct((NP, HID), _F32),
            jax.ShapeDtypeStruct((NP, 2 * HEADS), _F32),
            jax.ShapeDtypeStruct((NP, 2 * HEADS), _F32),
            jax.ShapeDtypeStruct((L,), _F32),
        ],
    )
    tc2 = pl.pallas_call(
        _tc2_body,
        out_shape=[
            jax.ShapeDtypeStruct((NP, CP), _F32),
            jax.ShapeDtypeStruct((NP, 1), _F32),
            jax.ShapeDtypeStruct((NP, 1), _F32),
            jax.ShapeDtypeStruct((L,), _F32),
        ],
    )
    tc3 = pl.pallas_call(
        _tc3_body,
        out_shape=jax.ShapeDtypeStruct((N, CLASSES), _F32),
    )
    mesh = plsc.VectorSubcoreMesh(
        core_axis_name="c", subcore_axis_name="s",
        num_cores=NC, num_subcores=NS)
    sc_params = pltpu.CompilerParams(use_tc_tiling_on_sc=False,
                                     needs_layout_passes=False)
    sc1 = pl.kernel(
        _sc1_body,
        out_type=[
            jax.ShapeDtypeStruct((NC, NP, 2 * HEADS), _F32),
            jax.ShapeDtypeStruct((NC, NP, HID), _F32),
        ],
        mesh=mesh,
        scratch_types=[
            pltpu.VMEM((NCH, B), _I32),
            pltpu.VMEM((NCH, B), _I32),
            pltpu.VMEM((B, 2 * HEADS), _F32),
            pltpu.VMEM((B, 2 * HEADS), _F32),
            pltpu.VMEM((B, 2 * HEADS), _F32),
            pltpu.VMEM((B, HID), _F32),
            pltpu.VMEM((B, HID), _F32),
            pltpu.VMEM((L,), _F32),
            pltpu.VMEM_SHARED((NP, 2 * HEADS), _F32),
            pltpu.VMEM_SHARED((NP, HID), _F32),
        ],
        compiler_params=sc_params,
    )
    sc2 = pl.kernel(
        _sc2_body,
        out_type=[
            jax.ShapeDtypeStruct((NC, NP), _F32),
            jax.ShapeDtypeStruct((NC, NP, CP), _F32),
        ],
        mesh=mesh,
        scratch_types=[
            pltpu.VMEM((NCH, B), _I32),
            pltpu.VMEM((NCH, B), _I32),
            pltpu.VMEM((NP,), _F32),
            pltpu.VMEM((NP,), _F32),
            pltpu.VMEM((B,), _F32),
            pltpu.VMEM((B, CP), _F32),
            pltpu.VMEM((B, CP), _F32),
            pltpu.VMEM((L,), _F32),
            pltpu.VMEM_SHARED((NP,), _F32),
            pltpu.VMEM_SHARED((NP, CP), _F32),
        ],
        compiler_params=sc_params,
    )
    return tc1, tc2, tc3, sc1, sc2


@jax.jit
def kernel(x, edges, W1, att_src1, att_dst1, bias1,
           W2, att_src2, att_dst2, bias2):
    _TC1, _TC2, _TC3, _SC1, _SC2 = _make_kernels()
    loop = jnp.arange(N, dtype=_I32)
    P = E2P - (edges.shape[1] + N)
    pad = jnp.arange(P, dtype=_I32)
    src = jnp.concatenate([edges[0].astype(_I32), loop, pad % N])
    dst = jnp.concatenate([edges[1].astype(_I32), loop, N + pad % (NP - N)])
    src3 = src.reshape(NW, NCH, B)
    dst3 = dst.reshape(NW, NCH, B)

    eye = jnp.eye(HEADS, dtype=_F32)
    amap_s = (att_src1[:, None, :, None] * eye[:, None, None, :]) \
        .reshape(HEADS, DH, HEADS).reshape(HID, HEADS)
    amap_d = (att_dst1[:, None, :, None] * eye[:, None, None, :]) \
        .reshape(HEADS, DH, HEADS).reshape(HID, HEADS)

    xp = jnp.pad(x, ((0, NP - N), (0, 0)))

    e8 = (jnp.eye(HEADS, dtype=_F32)[:, :, None] *
          jnp.ones((DH,), _F32)).reshape(HEADS, HID)

    h1, asd, add_, m1 = _TC1(xp, W1, amap_s, amap_d)
    den1, agg1 = _SC1(src3, dst3, asd, add_, h1, m1)
    h2p, as2, ad2, m2 = _TC2(den1, agg1, bias1, W2, att_src2, att_dst2, e8)
    den2, agg2 = _SC2(src3, dst3, as2.reshape(NP), ad2.reshape(NP), h2p, m2)
    return _TC3(den2, agg2, bias2)


# speedup vs baseline: 79.7900x; 1.4910x over previous
import functools

import jax
import jax.numpy as jnp
from jax import lax
from jax.experimental import pallas as pl
from jax.experimental.pallas import tpu as pltpu
from jax.experimental.pallas import tpu_sc as plsc

N = 10000
FEATS = 128
HID = 64
HEADS = 8
DH = 8
CLASSES = 40
CP = 48

NC = 2
NS = 16
NW = NC * NS
L = 16

NP = 10240
STRIPE = NP // NS
B = 128
NCH = 82
C = NCH * B
E2P = NW * C

_F32 = jnp.float32
_I32 = jnp.int32


def _iota16():
    return lax.iota(_I32, L)


def _vperm(v, idx):
    dn = lax.GatherDimensionNumbers(
        offset_dims=(), collapsed_slice_dims=(0,), start_index_map=(0,))
    return lax.gather(v, idx[:, None], dn, (1,),
                      mode=lax.GatherScatterMode.PROMISE_IN_BOUNDS)


def _tc1_body(xp_ref, w1_ref, amap_s_ref, amap_d_ref,
              h1_ref, asd_ref, add_ref, m1_ref):
    h = jnp.dot(xp_ref[...], w1_ref[...], preferred_element_type=_F32)
    h1_ref[...] = h
    a_s = jnp.dot(h, amap_s_ref[...], preferred_element_type=_F32)
    a_d = jnp.dot(h, amap_d_ref[...], preferred_element_type=_F32)
    asd_ref[...] = jnp.concatenate([a_s, a_s], axis=1)
    add_ref[...] = jnp.concatenate([a_d, a_d], axis=1)
    m = jnp.maximum(jnp.max(a_s, axis=0) + jnp.max(a_d, axis=0), 0.0)
    m1_ref[...] = jnp.concatenate([m, m], axis=0)


def _tc2_body(den_ref, out_ref, b1_ref, w2_ref, as2w_ref, ad2w_ref, e8_ref,
              h2p_ref, as2_ref, ad2_ref, m2_ref):
    den = den_ref[0, :, :HEADS] + den_ref[1, :, :HEADS]
    agg = out_ref[0] + out_ref[1]
    inv = 1.0 / (den + 1e-16)
    invx = jnp.dot(inv, e8_ref[...], preferred_element_type=_F32)
    h1f = agg * invx + b1_ref[...][None, :]
    rowid = lax.broadcasted_iota(_I32, (NP, 1), 0)
    h1f = jnp.where(rowid < N, h1f, 0.0)
    h2 = jnp.dot(h1f, w2_ref[...], preferred_element_type=_F32)
    h2p_ref[...] = jnp.pad(h2, ((0, 0), (0, CP - CLASSES)))
    a_s = jnp.dot(h2, as2w_ref[...].reshape(CLASSES, 1),
                  preferred_element_type=_F32)
    a_d = jnp.dot(h2, ad2w_ref[...].reshape(CLASSES, 1),
                  preferred_element_type=_F32)
    a_s = jnp.where(rowid < N, a_s, 0.0)
    a_d = jnp.where(rowid < N, a_d, 0.0)
    as2_ref[...] = a_s
    ad2_ref[...] = a_d
    m2 = jnp.maximum(jnp.max(a_s) + jnp.max(a_d), 0.0)
    m2_ref[...] = jnp.full((L,), m2, dtype=_F32)


def _tc3_body(den_ref, out_ref, b2_ref, y_ref):
    den = den_ref[0] + den_ref[1]
    agg = out_ref[0, :, :CLASSES] + out_ref[1, :, :CLASSES]
    inv = 1.0 / (den + 1e-16)
    y = agg * inv[:, None] + b2_ref[...][None, :]
    y_ref[...] = y[:N, :]


def _sc1_body(src_hbm, dst_hbm, asd_hbm, add_hbm, h1_hbm, m1_hbm,
              den_out, msg_out,
              sidx, didx, sbuf, dbuf, exb, hbuf, msgb, mvec,
              gsem, ssem, den_sh, out_sh):
    c = lax.axis_index("c")
    s = lax.axis_index("s")
    w = c * NS + s

    def _z16(i, _):
        exb[0, i, :] = jnp.zeros((L,), _F32)
        return 0
    lax.fori_loop(0, B, _z16, 0)

    def _z64(i, _):
        for q in range(4):
            msgb[0, i, pl.ds(q * L, L)] = jnp.zeros((L,), _F32)
        return 0
    lax.fori_loop(0, B, _z64, 0)

    for r in range(STRIPE // B):
        row = s * STRIPE + r * B
        pltpu.sync_copy(exb.at[0], den_sh.at[pl.ds(row, B)])
        pltpu.sync_copy(msgb.at[0], out_sh.at[pl.ds(row, B)])
    plsc.subcore_barrier()

    pltpu.sync_copy(src_hbm.at[w], sidx)
    pltpu.sync_copy(dst_hbm.at[w], didx)
    pltpu.sync_copy(m1_hbm, mvec)

    def _gathers(ch, b):
        pltpu.async_copy(asd_hbm.at[sidx.at[ch]], sbuf.at[b], gsem.at[b])
        pltpu.async_copy(add_hbm.at[didx.at[ch]], dbuf.at[b], gsem.at[b])
        pltpu.async_copy(h1_hbm.at[sidx.at[ch]], hbuf.at[b], gsem.at[b])

    def _gwait(ch, b):
        pltpu.make_async_copy(asd_hbm.at[sidx.at[ch]], sbuf.at[b],
                              gsem.at[b]).wait()
        pltpu.make_async_copy(add_hbm.at[didx.at[ch]], dbuf.at[b],
                              gsem.at[b]).wait()
        pltpu.make_async_copy(h1_hbm.at[sidx.at[ch]], hbuf.at[b],
                              gsem.at[b]).wait()

    def _swait(ch, b):
        pltpu.make_async_copy(exb.at[b], den_sh.at[didx.at[ch]],
                              ssem.at[b]).wait()
        pltpu.make_async_copy(msgb.at[b], out_sh.at[didx.at[ch]],
                              ssem.at[b]).wait()

    for b in range(2):
        _gathers(b, b)

    m = mvec[...]
    half = lax.shift_right_logical(_iota16(), 3)

    def _super(i, _):
        for b in range(2):
            ch = 2 * i + b
            _gwait(ch, b)

            @pl.when(i > 0)
            def _():
                _swait(ch - 2, b)

            def _edge(e2, _):
                for de in range(2):
                    e = 2 * e2 + de
                    t = sbuf[b, e, :] + dbuf[b, e, :]
                    a = jnp.maximum(t, 0.2 * t)
                    ex = jnp.exp(a - m)
                    exb[b, e, :] = ex
                    for q in range(4):
                        wv = _vperm(ex, half + (2 * q))
                        msgb[b, e, pl.ds(q * L, L)] = \
                            wv * hbuf[b, e, pl.ds(q * L, L)]
                return 0

            lax.fori_loop(0, B // 2, _edge, 0)
            di = didx.at[ch]
            pltpu.async_copy(exb.at[b], den_sh.at[di], ssem.at[b], add=True)
            pltpu.async_copy(msgb.at[b], out_sh.at[di], ssem.at[b], add=True)

            @pl.when(i < NCH // 2 - 1)
            def _():
                _gathers(ch + 2, b)
        return 0

    lax.fori_loop(0, NCH // 2, _super, 0)
    for b in range(2):
        _swait(NCH - 2 + b, b)
    plsc.subcore_barrier()

    row = s * STRIPE
    pltpu.sync_copy(den_sh.at[pl.ds(row, STRIPE)],
                    den_out.at[c].at[pl.ds(row, STRIPE)])
    pltpu.sync_copy(out_sh.at[pl.ds(row, STRIPE)],
                    msg_out.at[c].at[pl.ds(row, STRIPE)])


def _sc2_body(src_hbm, dst_hbm, as2_hbm, ad2_hbm, h2_hbm, m2_hbm,
              den_out, msg_out,
              sidx, didx, astab, adtab, exb, hbuf, msgb, mvec,
              gsem, ssem, den_sh, out_sh):
    c = lax.axis_index("c")
    s = lax.axis_index("s")
    w = c * NS + s

    def _z1(i, _):
        exb[0, pl.ds(i * L, L)] = jnp.zeros((L,), _F32)
        return 0
    lax.fori_loop(0, B // L, _z1, 0)

    def _z48(i, _):
        for q in range(3):
            msgb[0, i, pl.ds(q * L, L)] = jnp.zeros((L,), _F32)
        return 0
    lax.fori_loop(0, B, _z48, 0)

    for r in range(STRIPE // B):
        row = s * STRIPE + r * B
        pltpu.sync_copy(exb.at[0], den_sh.at[pl.ds(row, B)])
        pltpu.sync_copy(msgb.at[0], out_sh.at[pl.ds(row, B)])
    plsc.subcore_barrier()

    pltpu.sync_copy(src_hbm.at[w], sidx)
    pltpu.sync_copy(dst_hbm.at[w], didx)
    pltpu.sync_copy(m2_hbm, mvec)
    pltpu.sync_copy(as2_hbm, astab)
    pltpu.sync_copy(ad2_hbm, adtab)

    def _gwait(ch, b):
        pltpu.make_async_copy(h2_hbm.at[sidx.at[ch]], hbuf.at[b],
                              gsem.at[b]).wait()

    def _swait(ch, b):
        pltpu.make_async_copy(exb.at[b], den_sh.at[didx.at[ch]],
                              ssem.at[b]).wait()
        pltpu.make_async_copy(msgb.at[b], out_sh.at[didx.at[ch]],
                              ssem.at[b]).wait()

    for b in range(2):
        pltpu.async_copy(h2_hbm.at[sidx.at[b]], hbuf.at[b], gsem.at[b])

    m = mvec[...]

    def _super(i, _):
        for b in range(2):
            ch = 2 * i + b
            _gwait(ch, b)

            @pl.when(i > 0)
            def _():
                _swait(ch - 2, b)

            def _grp(g, _):
                sv = sidx[ch, pl.ds(g * L, L)]
                dv = didx[ch, pl.ds(g * L, L)]
                t = (plsc.load_gather(astab, [sv]) +
                     plsc.load_gather(adtab, [dv]))
                a = jnp.maximum(t, 0.2 * t)
                exg = jnp.exp(a - m)
                exb[b, pl.ds(g * L, L)] = exg

                def _edge(i2, _):
                    e = g * L + i2
                    wv = _vperm(exg, jnp.full((L,), i2, dtype=_I32))
                    for q in range(3):
                        msgb[b, e, pl.ds(q * L, L)] = \
                            wv * hbuf[b, e, pl.ds(q * L, L)]
                    return 0

                lax.fori_loop(0, L, _edge, 0)
                return 0

            lax.fori_loop(0, B // L, _grp, 0)
            di = didx.at[ch]
            pltpu.async_copy(exb.at[b], den_sh.at[di], ssem.at[b], add=True)
            pltpu.async_copy(msgb.at[b], out_sh.at[di], ssem.at[b], add=True)

            @pl.when(i < NCH // 2 - 1)
            def _():
                pltpu.async_copy(h2_hbm.at[sidx.at[ch + 2]], hbuf.at[b],
                                 gsem.at[b])
        return 0

    lax.fori_loop(0, NCH // 2, _super, 0)
    for b in range(2):
        _swait(NCH - 2 + b, b)
    plsc.subcore_barrier()

    row = s * STRIPE
    pltpu.sync_copy(den_sh.at[pl.ds(row, STRIPE)],
                    den_out.at[c].at[pl.ds(row, STRIPE)])
    pltpu.sync_copy(out_sh.at[pl.ds(row, STRIPE)],
                    msg_out.at[c].at[pl.ds(row, STRIPE)])


@functools.lru_cache(maxsize=1)
def _make_kernels():
    tc1 = pl.pallas_call(
        _tc1_body,
        out_shape=[
            jax.ShapeDtypeStruct((NP, HID), _F32),
            jax.ShapeDtypeStruct((NP, 2 * HEADS), _F32),
            jax.ShapeDtypeStruct((NP, 2 * HEADS), _F32),
            jax.ShapeDtypeStruct((L,), _F32),
        ],
    )
    tc2 = pl.pallas_call(
        _tc2_body,
        out_shape=[
            jax.ShapeDtypeStruct((NP, CP), _F32),
            jax.ShapeDtypeStruct((NP, 1), _F32),
            jax.ShapeDtypeStruct((NP, 1), _F32),
            jax.ShapeDtypeStruct((L,), _F32),
        ],
    )
    tc3 = pl.pallas_call(
        _tc3_body,
        out_shape=jax.ShapeDtypeStruct((N, CLASSES), _F32),
    )
    mesh = plsc.VectorSubcoreMesh(
        core_axis_name="c", subcore_axis_name="s",
        num_cores=NC, num_subcores=NS)
    sc_params = pltpu.CompilerParams(use_tc_tiling_on_sc=False,
                                     needs_layout_passes=False)
    sc1 = pl.kernel(
        _sc1_body,
        out_type=[
            jax.ShapeDtypeStruct((NC, NP, 2 * HEADS), _F32),
            jax.ShapeDtypeStruct((NC, NP, HID), _F32),
        ],
        mesh=mesh,
        scratch_types=[
            pltpu.VMEM((NCH, B), _I32),
            pltpu.VMEM((NCH, B), _I32),
            pltpu.VMEM((2, B, 2 * HEADS), _F32),
            pltpu.VMEM((2, B, 2 * HEADS), _F32),
            pltpu.VMEM((2, B, 2 * HEADS), _F32),
            pltpu.VMEM((2, B, HID), _F32),
            pltpu.VMEM((2, B, HID), _F32),
            pltpu.VMEM((L,), _F32),
            pltpu.SemaphoreType.DMA((2,)),
            pltpu.SemaphoreType.DMA((2,)),
            pltpu.VMEM_SHARED((NP, 2 * HEADS), _F32),
            pltpu.VMEM_SHARED((NP, HID), _F32),
        ],
        compiler_params=sc_params,
    )
    sc2 = pl.kernel(
        _sc2_body,
        out_type=[
            jax.ShapeDtypeStruct((NC, NP), _F32),
            jax.ShapeDtypeStruct((NC, NP, CP), _F32),
        ],
        mesh=mesh,
        scratch_types=[
            pltpu.VMEM((NCH, B), _I32),
            pltpu.VMEM((NCH, B), _I32),
            pltpu.VMEM((NP,), _F32),
            pltpu.VMEM((NP,), _F32),
            pltpu.VMEM((2, B), _F32),
            pltpu.VMEM((2, B, CP), _F32),
            pltpu.VMEM((2, B, CP), _F32),
            pltpu.VMEM((L,), _F32),
            pltpu.SemaphoreType.DMA((2,)),
            pltpu.SemaphoreType.DMA((2,)),
            pltpu.VMEM_SHARED((NP,), _F32),
            pltpu.VMEM_SHARED((NP, CP), _F32),
        ],
        compiler_params=sc_params,
    )
    return tc1, tc2, tc3, sc1, sc2


@jax.jit
def kernel(x, edges, W1, att_src1, att_dst1, bias1,
           W2, att_src2, att_dst2, bias2):
    _TC1, _TC2, _TC3, _SC1, _SC2 = _make_kernels()
    loop = jnp.arange(N, dtype=_I32)
    P = E2P - (edges.shape[1] + N)
    pad = jnp.arange(P, dtype=_I32)
    src = jnp.concatenate([edges[0].astype(_I32), loop, pad % N])
    dst = jnp.concatenate([edges[1].astype(_I32), loop, N + pad % (NP - N)])
    src3 = src.reshape(NW, NCH, B)
    dst3 = dst.reshape(NW, NCH, B)

    eye = jnp.eye(HEADS, dtype=_F32)
    amap_s = (att_src1[:, None, :, None] * eye[:, None, None, :]) \
        .reshape(HEADS, DH, HEADS).reshape(HID, HEADS)
    amap_d = (att_dst1[:, None, :, None] * eye[:, None, None, :]) \
        .reshape(HEADS, DH, HEADS).reshape(HID, HEADS)

    xp = jnp.pad(x, ((0, NP - N), (0, 0)))

    e8 = (jnp.eye(HEADS, dtype=_F32)[:, :, None] *
          jnp.ones((DH,), _F32)).reshape(HEADS, HID)

    h1, asd, add_, m1 = _TC1(xp, W1, amap_s, amap_d)
    den1, agg1 = _SC1(src3, dst3, asd, add_, h1, m1)
    h2p, as2, ad2, m2 = _TC2(den1, agg1, bias1, W2, att_src2, att_dst2, e8)
    den2, agg2 = _SC2(src3, dst3, as2.reshape(NP), ad2.reshape(NP), h2p, m2)
    return _TC3(den2, agg2, bias2)
